# Initial kernel scaffold; baseline (speedup 1.0000x reference)
import jax
import jax.numpy as jnp
from jax.experimental import pallas as pl


def kernel(x, edge_index, W_l, b_l, W_r, ln_gamma, ln_beta):
    raise NotImplementedError("write your pallas kernel here")


# speedup vs baseline: 6.3590x; 6.3590x over previous
import functools

import jax
import jax.numpy as jnp
from jax import lax
from jax.experimental import pallas as pl
from jax.experimental.pallas import tpu as pltpu
from jax.experimental.pallas import tpu_sc as plsc

N_NODES = 10000
N_EDGES = 320000
D = 128

_NC = 2
_NS = 16
_NW = _NC * _NS
_EPT = N_EDGES // _NW
_WIN = 80
_NWIN = _EPT // _WIN
_NPAD = 10240
_RPT = _NPAD // _NS


@functools.partial(
    pl.kernel,
    out_type=(
        jax.ShapeDtypeStruct((_NC, _NPAD, D), jnp.float32),
        jax.ShapeDtypeStruct((_NW, _NPAD), jnp.float32),
    ),
    mesh=plsc.VectorSubcoreMesh(core_axis_name="c", subcore_axis_name="s"),
    compiler_params=pltpu.CompilerParams(needs_layout_passes=False),
    scratch_types=[
        pltpu.VMEM((_WIN,), jnp.int32),
        pltpu.VMEM((_WIN,), jnp.int32),
        pltpu.VMEM((_WIN, D), jnp.float32),
        pltpu.VMEM((_NPAD,), jnp.float32),
        pltpu.VMEM_SHARED((_NPAD, D), jnp.float32),
        pltpu.SemaphoreType.DMA,
    ],
)
def _sc_segment_sum(x_hbm, src_hbm, dst_hbm, zeros_hbm, psum_hbm, cnt_hbm,
                    src_v, dst_v, rows_v, cnt_v, acc, sem):
    c = lax.axis_index("c")
    s = lax.axis_index("s")
    wid = c * _NS + s

    pltpu.sync_copy(zeros_hbm.at[pl.ds(s * _RPT, _RPT)],
                    acc.at[pl.ds(s * _RPT, _RPT)])

    zeros16 = jnp.zeros((16,), jnp.float32)

    def zbody(i, carry):
        cnt_v[pl.ds(i * 16, 16)] = zeros16
        return carry

    lax.fori_loop(0, _NPAD // 16, zbody, 0)
    plsc.subcore_barrier()

    base0 = wid * _EPT
    ones16 = jnp.ones((16,), jnp.float32)

    def window(w, carry):
        base = base0 + w * _WIN
        pltpu.sync_copy(src_hbm.at[pl.ds(base, _WIN)], src_v)
        pltpu.sync_copy(dst_hbm.at[pl.ds(base, _WIN)], dst_v)
        pltpu.async_copy(x_hbm.at[src_v], rows_v, sem).wait()
        pltpu.sync_copy(rows_v, acc.at[dst_v], add=True)
        for j in range(_WIN // 16):
            dvec = dst_v[pl.ds(j * 16, 16)]
            for lane in range(16):
                m = lax.iota(jnp.int32, 16) == lane
                plsc.addupdate_scatter(cnt_v, [dvec], ones16, mask=m)
        return carry

    lax.fori_loop(0, _NWIN, window, 0)

    plsc.subcore_barrier()
    pltpu.sync_copy(acc.at[pl.ds(s * _RPT, _RPT)],
                    psum_hbm.at[c, pl.ds(s * _RPT, _RPT)])
    pltpu.sync_copy(cnt_v, cnt_hbm.at[wid])


def _tc_body(psum_ref, cnts_ref, x_ref, wlt_ref, wrt_ref, bl_ref, g_ref,
             b_ref, o_ref):
    ones = jnp.ones((_NW, 1), jnp.float32)
    cnt = lax.dot_general(cnts_ref[...], ones, (((0,), (0,)), ((), ())),
                          preferred_element_type=jnp.float32)
    p = psum_ref[0] + psum_ref[1]
    agg = p / jnp.maximum(cnt, 1.0)
    h = (jnp.dot(agg, wlt_ref[...], preferred_element_type=jnp.float32)
         + jnp.dot(x_ref[...], wrt_ref[...], preferred_element_type=jnp.float32)
         + bl_ref[...])
    mean = jnp.mean(h, axis=1, keepdims=True)
    d = h - mean
    var = jnp.mean(d * d, axis=1, keepdims=True)
    hn = d * lax.rsqrt(var + 1e-5) * g_ref[...] + b_ref[...]
    o_ref[...] = 0.5 * hn * (1.0 + lax.erf(hn * 0.7071067811865476))


def _tc_epilogue(psum, cnts, x, wlt, wrt, bl, g, b):
    bn = 1024
    grid = ((N_NODES + bn - 1) // bn,)
    return pl.pallas_call(
        _tc_body,
        grid=grid,
        in_specs=[
            pl.BlockSpec((_NC, bn, D), lambda i: (0, i, 0)),
            pl.BlockSpec((_NW, bn), lambda i: (0, i)),
            pl.BlockSpec((bn, D), lambda i: (i, 0)),
            pl.BlockSpec((D, D), lambda i: (0, 0)),
            pl.BlockSpec((D, D), lambda i: (0, 0)),
            pl.BlockSpec((1, D), lambda i: (0, 0)),
            pl.BlockSpec((1, D), lambda i: (0, 0)),
            pl.BlockSpec((1, D), lambda i: (0, 0)),
        ],
        out_specs=pl.BlockSpec((bn, D), lambda i: (i, 0)),
        out_shape=jax.ShapeDtypeStruct((N_NODES, D), jnp.float32),
    )(psum, cnts, x, wlt, wrt, bl, g, b)


def kernel(x, edge_index, W_l, b_l, W_r, ln_gamma, ln_beta):
    ei = edge_index.astype(jnp.int32)
    src = ei[0]
    dst = ei[1]
    zeros = jnp.zeros((_NPAD, D), jnp.float32)
    psum, cnts = _sc_segment_sum(x, src, dst, zeros)
    return _tc_epilogue(psum, cnts, x, W_l.T, W_r.T,
                        b_l.reshape(1, D), ln_gamma.reshape(1, D),
                        ln_beta.reshape(1, D))


# speedup vs baseline: 11.2613x; 1.7709x over previous
import functools

import jax
import jax.numpy as jnp
from jax import lax
from jax.experimental import pallas as pl
from jax.experimental.pallas import tpu as pltpu
from jax.experimental.pallas import tpu_sc as plsc

N_NODES = 10000
N_EDGES = 320000
D = 128

_NC = 2
_NS = 16
_NW = _NC * _NS
_EPT = N_EDGES // _NW
_WIN = 80
_NWIN = _EPT // _WIN
_NPAD = 10240
_RPT = _NPAD // _NS


@functools.partial(
    pl.kernel,
    out_type=(
        jax.ShapeDtypeStruct((_NC, _NPAD, D), jnp.float32),
        jax.ShapeDtypeStruct((_NW, _NPAD), jnp.float32),
    ),
    mesh=plsc.VectorSubcoreMesh(core_axis_name="c", subcore_axis_name="s"),
    compiler_params=pltpu.CompilerParams(needs_layout_passes=False),
    scratch_types=[
        pltpu.VMEM((_WIN,), jnp.int32),
        pltpu.VMEM((_WIN,), jnp.int32),
        pltpu.VMEM((_WIN,), jnp.int32),
        pltpu.VMEM((_WIN,), jnp.int32),
        pltpu.VMEM((_WIN, D), jnp.float32),
        pltpu.VMEM((_WIN, D), jnp.float32),
        pltpu.VMEM((_NPAD,), jnp.float32),
        pltpu.VMEM_SHARED((_NPAD, D), jnp.float32),
        pltpu.SemaphoreType.DMA,
        pltpu.SemaphoreType.DMA,
    ],
)
def _sc_segment_sum(x_hbm, src_hbm, dst_hbm, zeros_hbm, psum_hbm, cnt_hbm,
                    src_v0, src_v1, dst_v0, dst_v1, rows_v0, rows_v1,
                    cnt_v, acc, semg, semi):
    c = lax.axis_index("c")
    s = lax.axis_index("s")
    wid = c * _NS + s
    src_v = (src_v0, src_v1)
    dst_v = (dst_v0, dst_v1)
    rows_v = (rows_v0, rows_v1)

    pltpu.sync_copy(zeros_hbm.at[pl.ds(s * _RPT, _RPT)],
                    acc.at[pl.ds(s * _RPT, _RPT)])

    zeros16 = jnp.zeros((16,), jnp.float32)

    def zbody(i, carry):
        cnt_v[pl.ds(i * 16, 16)] = zeros16
        return carry

    lax.fori_loop(0, _NPAD // 16, zbody, 0)
    plsc.subcore_barrier()

    base0 = wid * _EPT
    ones16 = jnp.ones((16,), jnp.float32)

    def stage_idx(w, p, sync=False):
        base = base0 + w * _WIN
        if sync:
            pltpu.sync_copy(src_hbm.at[pl.ds(base, _WIN)], src_v[p])
            pltpu.sync_copy(dst_hbm.at[pl.ds(base, _WIN)], dst_v[p])
        else:
            pltpu.async_copy(src_hbm.at[pl.ds(base, _WIN)], src_v[p], semi)
            pltpu.async_copy(dst_hbm.at[pl.ds(base, _WIN)], dst_v[p], semi)

    def wait_idx():
        pltpu.make_async_copy(src_hbm.at[pl.ds(base0, _WIN)], src_v0,
                              semi).wait()
        pltpu.make_async_copy(dst_hbm.at[pl.ds(base0, _WIN)], dst_v0,
                              semi).wait()

    def wait_gather(p):
        pltpu.make_async_copy(x_hbm.at[src_v[p]], rows_v[p], semg).wait()

    def hist(p):
        for j in range(_WIN // 16):
            dvec = dst_v[p][pl.ds(j * 16, 16)]
            for lane in range(16):
                m = lax.iota(jnp.int32, 16) == lane
                plsc.addupdate_scatter(cnt_v, [dvec], ones16, mask=m)

    def sub_iter(w, p):
        wait_gather(p)
        wait_idx()
        pltpu.async_copy(x_hbm.at[src_v[1 - p]], rows_v[1 - p], semg)
        hist(p)
        pltpu.sync_copy(rows_v[p], acc.at[dst_v[p]], add=True)

        @pl.when(w + 2 < _NWIN)
        def _():
            stage_idx(w + 2, p)

    stage_idx(0, 0, sync=True)
    pltpu.async_copy(x_hbm.at[src_v0], rows_v0, semg)
    stage_idx(1, 1)

    def pair(i, carry):
        sub_iter(2 * i, 0)
        sub_iter(2 * i + 1, 1)
        return carry

    lax.fori_loop(0, (_NWIN - 1) // 2, pair, 0)

    wait_gather(0)
    hist(0)
    pltpu.sync_copy(rows_v0, acc.at[dst_v0], add=True)

    plsc.subcore_barrier()
    pltpu.sync_copy(acc.at[pl.ds(s * _RPT, _RPT)],
                    psum_hbm.at[c, pl.ds(s * _RPT, _RPT)])
    pltpu.sync_copy(cnt_v, cnt_hbm.at[wid])


def _tc_body(psum_ref, cnts_ref, x_ref, wlt_ref, wrt_ref, bl_ref, g_ref,
             b_ref, o_ref):
    ones = jnp.ones((_NW, 1), jnp.float32)
    cnt = lax.dot_general(cnts_ref[...], ones, (((0,), (0,)), ((), ())),
                          preferred_element_type=jnp.float32)
    p = psum_ref[0] + psum_ref[1]
    agg = p / jnp.maximum(cnt, 1.0)
    h = (jnp.dot(agg, wlt_ref[...], preferred_element_type=jnp.float32)
         + jnp.dot(x_ref[...], wrt_ref[...], preferred_element_type=jnp.float32)
         + bl_ref[...])
    mean = jnp.mean(h, axis=1, keepdims=True)
    d = h - mean
    var = jnp.mean(d * d, axis=1, keepdims=True)
    hn = d * lax.rsqrt(var + 1e-5) * g_ref[...] + b_ref[...]
    o_ref[...] = 0.5 * hn * (1.0 + lax.erf(hn * 0.7071067811865476))


def _tc_epilogue(psum, cnts, x, wlt, wrt, bl, g, b):
    bn = 1024
    grid = ((N_NODES + bn - 1) // bn,)
    return pl.pallas_call(
        _tc_body,
        grid=grid,
        in_specs=[
            pl.BlockSpec((_NC, bn, D), lambda i: (0, i, 0)),
            pl.BlockSpec((_NW, bn), lambda i: (0, i)),
            pl.BlockSpec((bn, D), lambda i: (i, 0)),
            pl.BlockSpec((D, D), lambda i: (0, 0)),
            pl.BlockSpec((D, D), lambda i: (0, 0)),
            pl.BlockSpec((1, D), lambda i: (0, 0)),
            pl.BlockSpec((1, D), lambda i: (0, 0)),
            pl.BlockSpec((1, D), lambda i: (0, 0)),
        ],
        out_specs=pl.BlockSpec((bn, D), lambda i: (i, 0)),
        out_shape=jax.ShapeDtypeStruct((N_NODES, D), jnp.float32),
    )(psum, cnts, x, wlt, wrt, bl, g, b)


def kernel(x, edge_index, W_l, b_l, W_r, ln_gamma, ln_beta):
    ei = edge_index.astype(jnp.int32)
    src = ei[0]
    dst = ei[1]
    zeros = jnp.zeros((_NPAD, D), jnp.float32)
    psum, cnts = _sc_segment_sum(x, src, dst, zeros)
    return _tc_epilogue(psum, cnts, x, W_l.T, W_r.T,
                        b_l.reshape(1, D), ln_gamma.reshape(1, D),
                        ln_beta.reshape(1, D))


# speedup vs baseline: 11.3041x; 1.0038x over previous
import functools

import jax
import jax.numpy as jnp
from jax import lax
from jax.experimental import pallas as pl
from jax.experimental.pallas import tpu as pltpu
from jax.experimental.pallas import tpu_sc as plsc

N_NODES = 10000
N_EDGES = 320000
D = 128

_NC = 2
_NS = 16
_NW = _NC * _NS
_EPT = N_EDGES // _NW
_WIN = 80
_NWIN = _EPT // _WIN
_NPAD = 10240
_RPT = _NPAD // _NS


@functools.partial(
    pl.kernel,
    out_type=(
        jax.ShapeDtypeStruct((_NC, _NPAD, D), jnp.float32),
        jax.ShapeDtypeStruct((_NW, _NPAD), jnp.float32),
    ),
    mesh=plsc.VectorSubcoreMesh(core_axis_name="c", subcore_axis_name="s"),
    compiler_params=pltpu.CompilerParams(needs_layout_passes=False),
    scratch_types=[
        pltpu.VMEM((_WIN,), jnp.int32),
        pltpu.VMEM((_WIN,), jnp.int32),
        pltpu.VMEM((_WIN,), jnp.int32),
        pltpu.VMEM((_WIN,), jnp.int32),
        pltpu.VMEM((_WIN, D), jnp.float32),
        pltpu.VMEM((_WIN, D), jnp.float32),
        pltpu.VMEM((_NPAD,), jnp.float32),
        pltpu.VMEM_SHARED((_NPAD, D), jnp.float32),
        pltpu.SemaphoreType.DMA,
        pltpu.SemaphoreType.DMA,
    ],
)
def _sc_segment_sum(x_hbm, src_hbm, dst_hbm, zeros_hbm, psum_hbm, cnt_hbm,
                    src_v0, src_v1, dst_v0, dst_v1, rows_v0, rows_v1,
                    cnt_v, acc, semg, semi):
    c = lax.axis_index("c")
    s = lax.axis_index("s")
    wid = c * _NS + s
    src_v = (src_v0, src_v1)
    dst_v = (dst_v0, dst_v1)
    rows_v = (rows_v0, rows_v1)

    pltpu.sync_copy(zeros_hbm.at[pl.ds(s * _RPT, _RPT)],
                    acc.at[pl.ds(s * _RPT, _RPT)])

    zeros16 = jnp.zeros((16,), jnp.float32)

    def zbody(i, carry):
        cnt_v[pl.ds(i * 16, 16)] = zeros16
        return carry

    lax.fori_loop(0, _NPAD // 16, zbody, 0)
    plsc.subcore_barrier()

    base0 = wid * _EPT
    ones16 = jnp.ones((16,), jnp.float32)

    def stage_idx(w, p, sync=False):
        base = base0 + w * _WIN
        if sync:
            pltpu.sync_copy(src_hbm.at[pl.ds(base, _WIN)], src_v[p])
            pltpu.sync_copy(dst_hbm.at[pl.ds(base, _WIN)], dst_v[p])
        else:
            pltpu.async_copy(src_hbm.at[pl.ds(base, _WIN)], src_v[p], semi)
            pltpu.async_copy(dst_hbm.at[pl.ds(base, _WIN)], dst_v[p], semi)

    def wait_idx():
        pltpu.make_async_copy(src_hbm.at[pl.ds(base0, _WIN)], src_v0,
                              semi).wait()
        pltpu.make_async_copy(dst_hbm.at[pl.ds(base0, _WIN)], dst_v0,
                              semi).wait()

    def wait_gather(p):
        pltpu.make_async_copy(x_hbm.at[src_v[p]], rows_v[p], semg).wait()

    def hist(p):
        for j in range(_WIN // 16):
            dvec = dst_v[p][pl.ds(j * 16, 16)]
            plsc.addupdate_scatter(cnt_v, [dvec], ones16)

    def sub_iter(w, p):
        wait_gather(p)
        wait_idx()
        pltpu.async_copy(x_hbm.at[src_v[1 - p]], rows_v[1 - p], semg)
        hist(p)
        pltpu.sync_copy(rows_v[p], acc.at[dst_v[p]], add=True)

        @pl.when(w + 2 < _NWIN)
        def _():
            stage_idx(w + 2, p)

    stage_idx(0, 0, sync=True)
    pltpu.async_copy(x_hbm.at[src_v0], rows_v0, semg)
    stage_idx(1, 1)

    def pair(i, carry):
        sub_iter(2 * i, 0)
        sub_iter(2 * i + 1, 1)
        return carry

    lax.fori_loop(0, (_NWIN - 1) // 2, pair, 0)

    wait_gather(0)
    hist(0)
    pltpu.sync_copy(rows_v0, acc.at[dst_v0], add=True)

    plsc.subcore_barrier()
    pltpu.sync_copy(acc.at[pl.ds(s * _RPT, _RPT)],
                    psum_hbm.at[c, pl.ds(s * _RPT, _RPT)])
    pltpu.sync_copy(cnt_v, cnt_hbm.at[wid])


def _tc_body(psum_ref, cnts_ref, x_ref, wlt_ref, wrt_ref, bl_ref, g_ref,
             b_ref, o_ref):
    ones = jnp.ones((_NW, 1), jnp.float32)
    cnt = lax.dot_general(cnts_ref[...], ones, (((0,), (0,)), ((), ())),
                          preferred_element_type=jnp.float32)
    p = psum_ref[0] + psum_ref[1]
    agg = p / jnp.maximum(cnt, 1.0)
    h = (jnp.dot(agg, wlt_ref[...], preferred_element_type=jnp.float32)
         + jnp.dot(x_ref[...], wrt_ref[...], preferred_element_type=jnp.float32)
         + bl_ref[...])
    mean = jnp.mean(h, axis=1, keepdims=True)
    d = h - mean
    var = jnp.mean(d * d, axis=1, keepdims=True)
    hn = d * lax.rsqrt(var + 1e-5) * g_ref[...] + b_ref[...]
    o_ref[...] = 0.5 * hn * (1.0 + lax.erf(hn * 0.7071067811865476))


def _tc_epilogue(psum, cnts, x, wlt, wrt, bl, g, b):
    bn = 1024
    grid = ((N_NODES + bn - 1) // bn,)
    return pl.pallas_call(
        _tc_body,
        grid=grid,
        in_specs=[
            pl.BlockSpec((_NC, bn, D), lambda i: (0, i, 0)),
            pl.BlockSpec((_NW, bn), lambda i: (0, i)),
            pl.BlockSpec((bn, D), lambda i: (i, 0)),
            pl.BlockSpec((D, D), lambda i: (0, 0)),
            pl.BlockSpec((D, D), lambda i: (0, 0)),
            pl.BlockSpec((1, D), lambda i: (0, 0)),
            pl.BlockSpec((1, D), lambda i: (0, 0)),
            pl.BlockSpec((1, D), lambda i: (0, 0)),
        ],
        out_specs=pl.BlockSpec((bn, D), lambda i: (i, 0)),
        out_shape=jax.ShapeDtypeStruct((N_NODES, D), jnp.float32),
    )(psum, cnts, x, wlt, wrt, bl, g, b)


def kernel(x, edge_index, W_l, b_l, W_r, ln_gamma, ln_beta):
    ei = edge_index.astype(jnp.int32)
    src = ei[0]
    dst = ei[1]
    zeros = jnp.zeros((_NPAD, D), jnp.float32)
    psum, cnts = _sc_segment_sum(x, src, dst, zeros)
    return _tc_epilogue(psum, cnts, x, W_l.T, W_r.T,
                        b_l.reshape(1, D), ln_gamma.reshape(1, D),
                        ln_beta.reshape(1, D))
